# baseline (speedup 1.0000x reference)
import functools

import jax
import jax.numpy as jnp
from jax import lax
from jax.experimental import pallas as pl
from jax.experimental.pallas import tpu as pltpu
from jax.experimental.pallas import tpu_sc as plsc

N = 16384
K = 20
NUM_CLOUDS = 16
E = N * K

RB = 128
TW = 512
NBR_PAD = 128

_F32 = jnp.float32
_I32 = jnp.int32


def _knn_body(t0_ref, t1_ref, xt_ref, xq_ref, bcol_ref, brow_ref, nbr_ref,
              dist_scr, *, exact_d=0):
    g = pl.program_id(0)
    t0 = t0_ref[0, g]
    t1 = t1_ref[0, g]
    xq = xq_ref[...]
    brow = brow_ref[...]
    rows = g * RB + lax.broadcasted_iota(_I32, (RB, 1), 0)

    lane = lax.broadcasted_iota(_I32, (RB, NBR_PAD), 1)
    nbrs = jnp.zeros((RB, NBR_PAD), _I32)
    prev_k = jnp.full((RB, 1), -jnp.inf, _F32)
    prev_c = jnp.full((RB, 1), -1, _I32)
    big = jnp.int32(2 ** 30)

    nt = xt_ref.shape[0]
    for t in range(nt):
        @pl.when((t >= t0) & (t < t1))
        def _(t=t):
            c0 = t * TW
            xw = xt_ref[t]
            if exact_d:
                key = jnp.zeros((RB, TW), _F32)
                for dd in range(exact_d):
                    diff = xq[:, dd:dd + 1] - xw[dd:dd + 1, :]
                    key = key + diff * diff
            else:
                wn = jnp.sum(xw * xw, axis=0, keepdims=True)
                d = lax.dot_general(xq, xw, (((1,), (0,)), ((), ())),
                                    preferred_element_type=_F32,
                                    precision=lax.Precision.HIGHEST)
                key = wn - 2.0 * d
            cols = c0 + lax.broadcasted_iota(_I32, (RB, TW), 1)
            bw = bcol_ref[t]
            valid = (bw == brow) & (cols != rows)
            dist_scr[t] = jnp.where(valid, key, jnp.inf)

    for s in range(K):
        def sel_tile(t, carry, pk=prev_k, pc=prev_c):
            bk, bc = carry
            c0 = t * TW
            key = dist_scr[t]
            cols = c0 + lax.broadcasted_iota(_I32, (RB, TW), 1)
            newer = (key > pk) | ((key == pk) & (cols > pc))
            kk = jnp.where(newer, key, jnp.inf)
            mt = jnp.min(kk, axis=1, keepdims=True)
            ct = jnp.min(jnp.where(kk <= mt, cols, big), axis=1,
                         keepdims=True)
            take = (mt < bk) | ((mt == bk) & (ct < bc))
            return (jnp.where(take, mt, bk), jnp.where(take, ct, bc))

        bk0 = jnp.full((RB, 1), jnp.inf, _F32)
        bc0 = jnp.full((RB, 1), big, _I32)
        bk, bc = lax.fori_loop(t0, t1, sel_tile, (bk0, bc0))
        bc = jnp.where(bc >= big, 0, bc)
        nbrs = jnp.where(lane == s, bc, nbrs)
        prev_k, prev_c = bk, bc

    nbr_ref[...] = nbrs


def _knn(x, xt3, bcol3, brow, t0s, t1s, exact_d=0):
    n, d = x.shape
    nt = n // TW
    grid = n // RB
    return pl.pallas_call(
        functools.partial(_knn_body, exact_d=exact_d),
        grid=(grid,),
        in_specs=[
            pl.BlockSpec(memory_space=pltpu.SMEM),
            pl.BlockSpec(memory_space=pltpu.SMEM),
            pl.BlockSpec((nt, d, TW), lambda g: (0, 0, 0)),
            pl.BlockSpec((RB, d), lambda g: (g, 0)),
            pl.BlockSpec((nt, 1, TW), lambda g: (0, 0, 0)),
            pl.BlockSpec((RB, 1), lambda g: (g, 0)),
        ],
        out_specs=pl.BlockSpec((RB, NBR_PAD), lambda g: (g, 0)),
        out_shape=jax.ShapeDtypeStruct((n, NBR_PAD), _I32),
        scratch_shapes=[pltpu.VMEM((nt, RB, TW), _F32)],
    )(t0s, t1s, xt3, x, bcol3, brow)


def _sc_gather(table, idx):
    n, c = table.shape
    total = idx.shape[0]
    info = plsc.get_sparse_core_info()
    nc, ns = info.num_cores, info.num_subcores
    nw = nc * ns
    per_w = total // nw
    cb = 64 if c > 256 else 128
    n_chunks = per_w // cb
    mesh = plsc.VectorSubcoreMesh(core_axis_name="c", subcore_axis_name="s")

    @functools.partial(
        pl.kernel,
        mesh=mesh,
        out_type=jax.ShapeDtypeStruct((total, c), _F32),
        scratch_types=[
            pltpu.VMEM((cb,), _I32),
            pltpu.VMEM((cb, c), _F32),
            pltpu.SemaphoreType.DMA,
        ],
    )
    def gather_k(table_hbm, idx_hbm, out_hbm, idx_v, rows_v, sem):
        wid = lax.axis_index("s") * nc + lax.axis_index("c")
        base = wid * per_w

        def chunk(ci, _):
            off = base + ci * cb
            pltpu.sync_copy(idx_hbm.at[pl.ds(off, cb)], idx_v)
            pltpu.async_copy(table_hbm.at[idx_v], rows_v, sem).wait()
            pltpu.sync_copy(rows_v, out_hbm.at[pl.ds(off, cb)])
            return 0

        lax.fori_loop(0, n_chunks, chunk, 0, unroll=False)

    return gather_k(table, idx)


def _edge_h(x, xg3_kk, wcat_ref, b):
    ef = jnp.concatenate([x - xg3_kk, xg3_kk], axis=1)
    h = lax.dot_general(ef, wcat_ref[...], (((1,), (0,)), ((), ())),
                        preferred_element_type=_F32)
    return h + b


def _stats_body(x_ref, xg_ref, w_ref, b_ref, o_ref):
    g = pl.program_id(0)
    x = x_ref[...]
    b = b_ref[...]
    c = b.shape[1]
    s_acc = jnp.zeros((1, c), _F32)
    q_acc = jnp.zeros((1, c), _F32)
    for kk in range(K):
        h = _edge_h(x, xg_ref[kk], w_ref, b)
        s_acc = s_acc + jnp.sum(h, axis=0, keepdims=True)
        q_acc = q_acc + jnp.sum(h * h, axis=0, keepdims=True)
    upd = jnp.concatenate([s_acc, q_acc, jnp.zeros((6, c), _F32)], axis=0)

    @pl.when(g == 0)
    def _():
        o_ref[...] = jnp.zeros_like(o_ref)

    o_ref[...] += upd


def _stats(x, xg3, wcat, b_row, blk=128):
    n, dp = x.shape
    c = wcat.shape[1]
    return pl.pallas_call(
        _stats_body,
        grid=(n // blk,),
        in_specs=[
            pl.BlockSpec((blk, dp), lambda g: (g, 0)),
            pl.BlockSpec((K, blk, dp), lambda g: (0, g, 0)),
            pl.BlockSpec((2 * dp, c), lambda g: (0, 0)),
            pl.BlockSpec((1, c), lambda g: (0, 0)),
        ],
        out_specs=pl.BlockSpec((8, c), lambda g: (0, 0)),
        out_shape=jax.ShapeDtypeStruct((8, c), _F32),
    )(x, xg3, wcat, b_row)


def _out_body(x_ref, xg_ref, w_ref, b_ref, a_ref, b0_ref, o_ref):
    x = x_ref[...]
    b = b_ref[...]
    a = a_ref[...]
    b0 = b0_ref[...]
    acc = jnp.zeros((x.shape[0], b.shape[1]), _F32)
    for kk in range(K):
        h = _edge_h(x, xg_ref[kk], w_ref, b)
        acc = acc + jnp.maximum(a * h + b0, 0.0)
    o_ref[...] = acc


def _edge_out(x, xg3, wcat, b_row, a_row, b0_row, blk=128):
    n, dp = x.shape
    c = wcat.shape[1]
    return pl.pallas_call(
        _out_body,
        grid=(n // blk,),
        in_specs=[
            pl.BlockSpec((blk, dp), lambda g: (g, 0)),
            pl.BlockSpec((K, blk, dp), lambda g: (0, g, 0)),
            pl.BlockSpec((2 * dp, c), lambda g: (0, 0)),
            pl.BlockSpec((1, c), lambda g: (0, 0)),
            pl.BlockSpec((1, c), lambda g: (0, 0)),
            pl.BlockSpec((1, c), lambda g: (0, 0)),
        ],
        out_specs=pl.BlockSpec((blk, c), lambda g: (g, 0)),
        out_shape=jax.ShapeDtypeStruct((n, c), _F32),
    )(x, xg3, wcat, b_row, a_row, b0_row)


def _final_body(x_ref, b_ref, w1_ref, b1_ref, w2_ref, b2_ref, o_ref,
                pool_scr):
    g = pl.program_id(0)
    ng = pl.num_programs(0)
    x = x_ref[...]
    brow = b_ref[...]

    parts = []
    for cl in range(NUM_CLOUDS):
        m = jnp.where(brow == cl, x, -jnp.inf)
        parts.append(jnp.max(m, axis=0, keepdims=True))
    upd = jnp.concatenate(parts, axis=0)

    @pl.when(g == 0)
    def _():
        pool_scr[...] = jnp.full_like(pool_scr, -jnp.inf)

    pool_scr[...] = jnp.maximum(pool_scr[...], upd)

    @pl.when(g == ng - 1)
    def _():
        pooled = pool_scr[...]
        h = lax.dot_general(pooled, w1_ref[...], (((1,), (0,)), ((), ())),
                            preferred_element_type=_F32)
        h = jnp.maximum(h + b1_ref[...], 0.0)
        o = lax.dot_general(h, w2_ref[...], (((1,), (0,)), ((), ())),
                            preferred_element_type=_F32)
        o_ref[...] = o + b2_ref[...]


def _final(x4, brow, w1t, b1_row, w2t, b2_row, blk=512):
    n, c = x4.shape
    co = w2t.shape[1]
    return pl.pallas_call(
        _final_body,
        grid=(n // blk,),
        in_specs=[
            pl.BlockSpec((blk, c), lambda g: (g, 0)),
            pl.BlockSpec((blk, 1), lambda g: (g, 0)),
            pl.BlockSpec((c, w1t.shape[1]), lambda g: (0, 0)),
            pl.BlockSpec((1, w1t.shape[1]), lambda g: (0, 0)),
            pl.BlockSpec((w2t.shape[0], co), lambda g: (0, 0)),
            pl.BlockSpec((1, co), lambda g: (0, 0)),
        ],
        out_specs=pl.BlockSpec((NUM_CLOUDS, co), lambda g: (0, 0)),
        out_shape=jax.ShapeDtypeStruct((NUM_CLOUDS, co), _F32),
        scratch_shapes=[pltpu.VMEM((NUM_CLOUDS, c), _F32)],
    )(x4, brow, w1t, b1_row, w2t, b2_row)


def _edge_conv(x, bcol, brow, t0s, t1s, W, b, gamma, beta):
    n, d_raw = x.shape
    c = W.shape[0]
    d = max(8, d_raw)
    if d_raw < d:
        x = jnp.pad(x, ((0, 0), (0, d - d_raw)))
    dp = 128
    xp = jnp.pad(x, ((0, 0), (0, dp - d))) if d < dp else x
    wa = W[:, :d_raw]
    wb = W[:, d_raw:]
    wcat = jnp.zeros((2 * dp, c), _F32)
    wcat = wcat.at[:d_raw, :].set(wa.T)
    wcat = wcat.at[dp:dp + d_raw, :].set(wb.T)

    xt3 = x.T.reshape(d, n // TW, TW).transpose(1, 0, 2)
    nbr_pad = _knn(x, xt3, bcol, brow, t0s, t1s,
                   exact_d=d_raw if d_raw <= 8 else 0)

    idx = nbr_pad[:, :K].T.reshape(E)
    xg = _sc_gather(xp, idx)
    xg3 = xg.reshape(K, N, dp)

    b_row = b.reshape(1, c)
    st = _stats(xp, xg3, wcat, b_row)
    mean = st[0] / E
    var = st[1] / E - mean * mean
    scale = gamma / jnp.sqrt(var + 1e-5)
    a_row = scale.reshape(1, c)
    b0_row = (beta - scale * mean).reshape(1, c)
    return _edge_out(xp, xg3, wcat, b_row, a_row, b0_row)


def kernel(pos, batch, W1, b1, g1, be1, W2, b2, g2, be2, W3, b3, g3, be3,
           W4, b4, g4, be4, Wc1, bc1, Wc2, bc2):
    batch = batch.astype(_I32)
    bcol = batch.reshape(N // TW, 1, TW)
    brow = batch.reshape(N, 1)

    starts = jnp.searchsorted(batch, jnp.arange(NUM_CLOUDS + 1, dtype=_I32))
    blk_lo = batch[::RB]
    blk_hi = batch[RB - 1::RB]
    w0 = starts[blk_lo]
    w1 = starts[blk_hi + 1]
    t0s = (w0 // TW).reshape(1, -1).astype(_I32)
    t1s = ((w1 + TW - 1) // TW).reshape(1, -1).astype(_I32)

    x = _edge_conv(pos, bcol, brow, t0s, t1s, W1, b1, g1, be1)
    x = _edge_conv(x, bcol, brow, t0s, t1s, W2, b2, g2, be2)
    x = _edge_conv(x, bcol, brow, t0s, t1s, W3, b3, g3, be3)
    x = _edge_conv(x, bcol, brow, t0s, t1s, W4, b4, g4, be4)

    co_pad = 128
    w2t = jnp.zeros((Wc1.shape[0], co_pad), _F32).at[:, :Wc2.shape[0]].set(Wc2.T)
    b2_row = jnp.zeros((1, co_pad), _F32).at[0, :Wc2.shape[0]].set(bc2)
    out_pad = _final(x, brow, Wc1.T, bc1.reshape(1, -1), w2t, b2_row)
    return out_pad[:, :Wc2.shape[0]]


# speedup vs baseline: 1.1709x; 1.1709x over previous
import functools

import jax
import jax.numpy as jnp
from jax import lax
from jax.experimental import pallas as pl
from jax.experimental.pallas import tpu as pltpu
from jax.experimental.pallas import tpu_sc as plsc

N = 16384
K = 20
NUM_CLOUDS = 16
E = N * K

RB = 512
TW = 512
NBR_PAD = 128

_F32 = jnp.float32
_I32 = jnp.int32


def _knn_body(t0_ref, t1_ref, xt_ref, xq_ref, bcol_ref, brow_ref, nbr_ref,
              dist_scr, *, exact_d=0):
    g = pl.program_id(0)
    t0 = t0_ref[0, g]
    t1 = t1_ref[0, g]
    xq = xq_ref[...]
    brow = brow_ref[...]
    rows = g * RB + lax.broadcasted_iota(_I32, (RB, 1), 0)

    lane = lax.broadcasted_iota(_I32, (RB, NBR_PAD), 1)
    nbrs = jnp.zeros((RB, NBR_PAD), _I32)
    prev_k = jnp.full((RB, 1), -jnp.inf, _F32)
    prev_c = jnp.full((RB, 1), -1, _I32)
    big = jnp.int32(2 ** 30)

    nt = xt_ref.shape[0]
    for t in range(nt):
        @pl.when((t >= t0) & (t < t1))
        def _(t=t):
            c0 = t * TW
            xw = xt_ref[t]
            if exact_d:
                key = jnp.zeros((RB, TW), _F32)
                for dd in range(exact_d):
                    diff = xq[:, dd:dd + 1] - xw[dd:dd + 1, :]
                    key = key + diff * diff
            else:
                wn = jnp.sum(xw * xw, axis=0, keepdims=True)
                d = lax.dot_general(xq, xw, (((1,), (0,)), ((), ())),
                                    preferred_element_type=_F32,
                                    precision=lax.Precision.HIGHEST)
                key = wn - 2.0 * d
            cols = c0 + lax.broadcasted_iota(_I32, (RB, TW), 1)
            bw = bcol_ref[t]
            valid = (bw == brow) & (cols != rows)
            dist_scr[t] = jnp.where(valid, key, jnp.inf)

    for s in range(K):
        def sel_tile(t, carry, pk=prev_k, pc=prev_c):
            bk, bc = carry
            c0 = t * TW
            key = dist_scr[t]
            cols = c0 + lax.broadcasted_iota(_I32, (RB, TW), 1)
            newer = (key > pk) | ((key == pk) & (cols > pc))
            kk = jnp.where(newer, key, jnp.inf)
            mt = jnp.min(kk, axis=1, keepdims=True)
            ct = jnp.min(jnp.where(kk <= mt, cols, big), axis=1,
                         keepdims=True)
            take = (mt < bk) | ((mt == bk) & (ct < bc))
            return (jnp.where(take, mt, bk), jnp.where(take, ct, bc))

        bk0 = jnp.full((RB, 1), jnp.inf, _F32)
        bc0 = jnp.full((RB, 1), big, _I32)
        bk, bc = lax.fori_loop(t0, t1, sel_tile, (bk0, bc0))
        bc = jnp.where(bc >= big, 0, bc)
        nbrs = jnp.where(lane == s, bc, nbrs)
        prev_k, prev_c = bk, bc

    nbr_ref[...] = nbrs


def _knn(x, xt3, bcol3, brow, t0s, t1s, exact_d=0):
    n, d = x.shape
    nt = n // TW
    grid = n // RB
    return pl.pallas_call(
        functools.partial(_knn_body, exact_d=exact_d),
        grid=(grid,),
        in_specs=[
            pl.BlockSpec(memory_space=pltpu.SMEM),
            pl.BlockSpec(memory_space=pltpu.SMEM),
            pl.BlockSpec((nt, d, TW), lambda g: (0, 0, 0)),
            pl.BlockSpec((RB, d), lambda g: (g, 0)),
            pl.BlockSpec((nt, 1, TW), lambda g: (0, 0, 0)),
            pl.BlockSpec((RB, 1), lambda g: (g, 0)),
        ],
        out_specs=pl.BlockSpec((RB, NBR_PAD), lambda g: (g, 0)),
        out_shape=jax.ShapeDtypeStruct((n, NBR_PAD), _I32),
        scratch_shapes=[pltpu.VMEM((nt, RB, TW), _F32)],
    )(t0s, t1s, xt3, x, bcol3, brow)


def _sc_gather(table, idx):
    n, c = table.shape
    total = idx.shape[0]
    info = plsc.get_sparse_core_info()
    nc, ns = info.num_cores, info.num_subcores
    nw = nc * ns
    per_w = total // nw
    cb = 64 if c > 256 else 128
    n_chunks = per_w // cb
    mesh = plsc.VectorSubcoreMesh(core_axis_name="c", subcore_axis_name="s")

    @functools.partial(
        pl.kernel,
        mesh=mesh,
        out_type=jax.ShapeDtypeStruct((total, c), _F32),
        scratch_types=[
            pltpu.VMEM((cb,), _I32),
            pltpu.VMEM((cb, c), _F32),
            pltpu.SemaphoreType.DMA,
        ],
    )
    def gather_k(table_hbm, idx_hbm, out_hbm, idx_v, rows_v, sem):
        wid = lax.axis_index("s") * nc + lax.axis_index("c")
        base = wid * per_w

        def chunk(ci, _):
            off = base + ci * cb
            pltpu.sync_copy(idx_hbm.at[pl.ds(off, cb)], idx_v)
            pltpu.async_copy(table_hbm.at[idx_v], rows_v, sem).wait()
            pltpu.sync_copy(rows_v, out_hbm.at[pl.ds(off, cb)])
            return 0

        lax.fori_loop(0, n_chunks, chunk, 0, unroll=False)

    return gather_k(table, idx)


def _edge_h(x, xg3_kk, wcat_ref, b):
    ef = jnp.concatenate([x - xg3_kk, xg3_kk], axis=1)
    h = lax.dot_general(ef, wcat_ref[...], (((1,), (0,)), ((), ())),
                        preferred_element_type=_F32)
    return h + b


def _stats_body(x_ref, xg_ref, w_ref, b_ref, o_ref):
    g = pl.program_id(0)
    x = x_ref[...]
    b = b_ref[...]
    c = b.shape[1]
    s_acc = jnp.zeros((1, c), _F32)
    q_acc = jnp.zeros((1, c), _F32)
    for kk in range(K):
        h = _edge_h(x, xg_ref[kk], w_ref, b)
        s_acc = s_acc + jnp.sum(h, axis=0, keepdims=True)
        q_acc = q_acc + jnp.sum(h * h, axis=0, keepdims=True)
    upd = jnp.concatenate([s_acc, q_acc, jnp.zeros((6, c), _F32)], axis=0)

    @pl.when(g == 0)
    def _():
        o_ref[...] = jnp.zeros_like(o_ref)

    o_ref[...] += upd


def _stats(x, xg3, wcat, b_row, blk=128):
    n, dp = x.shape
    c = wcat.shape[1]
    return pl.pallas_call(
        _stats_body,
        grid=(n // blk,),
        in_specs=[
            pl.BlockSpec((blk, dp), lambda g: (g, 0)),
            pl.BlockSpec((K, blk, dp), lambda g: (0, g, 0)),
            pl.BlockSpec((2 * dp, c), lambda g: (0, 0)),
            pl.BlockSpec((1, c), lambda g: (0, 0)),
        ],
        out_specs=pl.BlockSpec((8, c), lambda g: (0, 0)),
        out_shape=jax.ShapeDtypeStruct((8, c), _F32),
    )(x, xg3, wcat, b_row)


def _out_body(x_ref, xg_ref, w_ref, b_ref, a_ref, b0_ref, o_ref):
    x = x_ref[...]
    b = b_ref[...]
    a = a_ref[...]
    b0 = b0_ref[...]
    acc = jnp.zeros((x.shape[0], b.shape[1]), _F32)
    for kk in range(K):
        h = _edge_h(x, xg_ref[kk], w_ref, b)
        acc = acc + jnp.maximum(a * h + b0, 0.0)
    o_ref[...] = acc


def _edge_out(x, xg3, wcat, b_row, a_row, b0_row, blk=128):
    n, dp = x.shape
    c = wcat.shape[1]
    return pl.pallas_call(
        _out_body,
        grid=(n // blk,),
        in_specs=[
            pl.BlockSpec((blk, dp), lambda g: (g, 0)),
            pl.BlockSpec((K, blk, dp), lambda g: (0, g, 0)),
            pl.BlockSpec((2 * dp, c), lambda g: (0, 0)),
            pl.BlockSpec((1, c), lambda g: (0, 0)),
            pl.BlockSpec((1, c), lambda g: (0, 0)),
            pl.BlockSpec((1, c), lambda g: (0, 0)),
        ],
        out_specs=pl.BlockSpec((blk, c), lambda g: (g, 0)),
        out_shape=jax.ShapeDtypeStruct((n, c), _F32),
    )(x, xg3, wcat, b_row, a_row, b0_row)


def _final_body(x_ref, b_ref, w1_ref, b1_ref, w2_ref, b2_ref, o_ref,
                pool_scr):
    g = pl.program_id(0)
    ng = pl.num_programs(0)
    x = x_ref[...]
    brow = b_ref[...]

    parts = []
    for cl in range(NUM_CLOUDS):
        m = jnp.where(brow == cl, x, -jnp.inf)
        parts.append(jnp.max(m, axis=0, keepdims=True))
    upd = jnp.concatenate(parts, axis=0)

    @pl.when(g == 0)
    def _():
        pool_scr[...] = jnp.full_like(pool_scr, -jnp.inf)

    pool_scr[...] = jnp.maximum(pool_scr[...], upd)

    @pl.when(g == ng - 1)
    def _():
        pooled = pool_scr[...]
        h = lax.dot_general(pooled, w1_ref[...], (((1,), (0,)), ((), ())),
                            preferred_element_type=_F32)
        h = jnp.maximum(h + b1_ref[...], 0.0)
        o = lax.dot_general(h, w2_ref[...], (((1,), (0,)), ((), ())),
                            preferred_element_type=_F32)
        o_ref[...] = o + b2_ref[...]


def _final(x4, brow, w1t, b1_row, w2t, b2_row, blk=512):
    n, c = x4.shape
    co = w2t.shape[1]
    return pl.pallas_call(
        _final_body,
        grid=(n // blk,),
        in_specs=[
            pl.BlockSpec((blk, c), lambda g: (g, 0)),
            pl.BlockSpec((blk, 1), lambda g: (g, 0)),
            pl.BlockSpec((c, w1t.shape[1]), lambda g: (0, 0)),
            pl.BlockSpec((1, w1t.shape[1]), lambda g: (0, 0)),
            pl.BlockSpec((w2t.shape[0], co), lambda g: (0, 0)),
            pl.BlockSpec((1, co), lambda g: (0, 0)),
        ],
        out_specs=pl.BlockSpec((NUM_CLOUDS, co), lambda g: (0, 0)),
        out_shape=jax.ShapeDtypeStruct((NUM_CLOUDS, co), _F32),
        scratch_shapes=[pltpu.VMEM((NUM_CLOUDS, c), _F32)],
    )(x4, brow, w1t, b1_row, w2t, b2_row)


def _edge_conv(x, bcol, brow, t0s, t1s, W, b, gamma, beta):
    n, d_raw = x.shape
    c = W.shape[0]
    d = max(8, d_raw)
    if d_raw < d:
        x = jnp.pad(x, ((0, 0), (0, d - d_raw)))
    dp = 128
    xp = jnp.pad(x, ((0, 0), (0, dp - d))) if d < dp else x
    wa = W[:, :d_raw]
    wb = W[:, d_raw:]
    wcat = jnp.zeros((2 * dp, c), _F32)
    wcat = wcat.at[:d_raw, :].set(wa.T)
    wcat = wcat.at[dp:dp + d_raw, :].set(wb.T)

    xt3 = x.T.reshape(d, n // TW, TW).transpose(1, 0, 2)
    nbr_pad = _knn(x, xt3, bcol, brow, t0s, t1s,
                   exact_d=d_raw if d_raw <= 8 else 0)

    idx = nbr_pad[:, :K].T.reshape(E)
    xg = _sc_gather(xp, idx)
    xg3 = xg.reshape(K, N, dp)

    b_row = b.reshape(1, c)
    st = _stats(xp, xg3, wcat, b_row)
    mean = st[0] / E
    var = st[1] / E - mean * mean
    scale = gamma / jnp.sqrt(var + 1e-5)
    a_row = scale.reshape(1, c)
    b0_row = (beta - scale * mean).reshape(1, c)
    return _edge_out(xp, xg3, wcat, b_row, a_row, b0_row)


def kernel(pos, batch, W1, b1, g1, be1, W2, b2, g2, be2, W3, b3, g3, be3,
           W4, b4, g4, be4, Wc1, bc1, Wc2, bc2):
    batch = batch.astype(_I32)
    bcol = batch.reshape(N // TW, 1, TW)
    brow = batch.reshape(N, 1)

    starts = jnp.searchsorted(batch, jnp.arange(NUM_CLOUDS + 1, dtype=_I32))
    blk_lo = batch[::RB]
    blk_hi = batch[RB - 1::RB]
    w0 = starts[blk_lo]
    w1 = starts[blk_hi + 1]
    t0s = (w0 // TW).reshape(1, -1).astype(_I32)
    t1s = ((w1 + TW - 1) // TW).reshape(1, -1).astype(_I32)

    x = _edge_conv(pos, bcol, brow, t0s, t1s, W1, b1, g1, be1)
    x = _edge_conv(x, bcol, brow, t0s, t1s, W2, b2, g2, be2)
    x = _edge_conv(x, bcol, brow, t0s, t1s, W3, b3, g3, be3)
    x = _edge_conv(x, bcol, brow, t0s, t1s, W4, b4, g4, be4)

    co_pad = 128
    w2t = jnp.zeros((Wc1.shape[0], co_pad), _F32).at[:, :Wc2.shape[0]].set(Wc2.T)
    b2_row = jnp.zeros((1, co_pad), _F32).at[0, :Wc2.shape[0]].set(bc2)
    out_pad = _final(x, brow, Wc1.T, bc1.reshape(1, -1), w2t, b2_row)
    return out_pad[:, :Wc2.shape[0]]


# speedup vs baseline: 1.4246x; 1.2167x over previous
import functools

import jax
import jax.numpy as jnp
from jax import lax
from jax.experimental import pallas as pl
from jax.experimental.pallas import tpu as pltpu
from jax.experimental.pallas import tpu_sc as plsc

N = 16384
K = 20
NUM_CLOUDS = 16
E = N * K

RB = 256
TW = 1024
NBR_PAD = 128

_F32 = jnp.float32
_I32 = jnp.int32


def _knn_body(t0_ref, t1_ref, xt_ref, xq_ref, bcol_ref, brow_ref, nbr_ref,
              dist_scr, *, exact_d=0):
    g = pl.program_id(0)
    t0 = t0_ref[0, g]
    t1 = t1_ref[0, g]
    xq = xq_ref[...]
    brow = brow_ref[...]
    rows = g * RB + lax.broadcasted_iota(_I32, (RB, 1), 0)

    lane = lax.broadcasted_iota(_I32, (RB, NBR_PAD), 1)
    nbrs = jnp.zeros((RB, NBR_PAD), _I32)
    prev_k = jnp.full((RB, 1), -jnp.inf, _F32)
    prev_c = jnp.full((RB, 1), -1, _I32)
    big = jnp.int32(2 ** 30)

    nt = xt_ref.shape[0]
    for t in range(nt):
        @pl.when((t >= t0) & (t < t1))
        def _(t=t):
            c0 = t * TW
            xw = xt_ref[t]
            if exact_d:
                key = jnp.zeros((RB, TW), _F32)
                for dd in range(exact_d):
                    diff = xq[:, dd:dd + 1] - xw[dd:dd + 1, :]
                    key = key + diff * diff
            else:
                wn = jnp.sum(xw * xw, axis=0, keepdims=True)
                d = lax.dot_general(xq, xw, (((1,), (0,)), ((), ())),
                                    preferred_element_type=_F32,
                                    precision=lax.Precision.HIGHEST)
                key = wn - 2.0 * d
            cols = c0 + lax.broadcasted_iota(_I32, (RB, TW), 1)
            bw = bcol_ref[t]
            valid = (bw == brow) & (cols != rows)
            dist_scr[t] = jnp.where(valid, key, jnp.inf)

    for s in range(K):
        def sel_tile(t, carry, pk=prev_k, pc=prev_c):
            bk, bc = carry
            c0 = t * TW
            key = dist_scr[t]
            cols = c0 + lax.broadcasted_iota(_I32, (RB, TW), 1)
            newer = (key > pk) | ((key == pk) & (cols > pc))
            kk = jnp.where(newer, key, jnp.inf)
            mt = jnp.min(kk, axis=1, keepdims=True)
            ct = jnp.min(jnp.where(kk <= mt, cols, big), axis=1,
                         keepdims=True)
            take = (mt < bk) | ((mt == bk) & (ct < bc))
            return (jnp.where(take, mt, bk), jnp.where(take, ct, bc))

        bk0 = jnp.full((RB, 1), jnp.inf, _F32)
        bc0 = jnp.full((RB, 1), big, _I32)
        bk, bc = lax.fori_loop(t0, t1, sel_tile, (bk0, bc0))
        bc = jnp.where(bc >= big, 0, bc)
        nbrs = jnp.where(lane == s, bc, nbrs)
        prev_k, prev_c = bk, bc

    nbr_ref[...] = nbrs


def _knn(x, xt3, bcol3, brow, t0s, t1s, exact_d=0):
    n, d = x.shape
    nt = n // TW
    grid = n // RB
    return pl.pallas_call(
        functools.partial(_knn_body, exact_d=exact_d),
        grid=(grid,),
        in_specs=[
            pl.BlockSpec(memory_space=pltpu.SMEM),
            pl.BlockSpec(memory_space=pltpu.SMEM),
            pl.BlockSpec((nt, d, TW), lambda g: (0, 0, 0)),
            pl.BlockSpec((RB, d), lambda g: (g, 0)),
            pl.BlockSpec((nt, 1, TW), lambda g: (0, 0, 0)),
            pl.BlockSpec((RB, 1), lambda g: (g, 0)),
        ],
        out_specs=pl.BlockSpec((RB, NBR_PAD), lambda g: (g, 0)),
        out_shape=jax.ShapeDtypeStruct((n, NBR_PAD), _I32),
        scratch_shapes=[pltpu.VMEM((nt, RB, TW), _F32)],
    )(t0s, t1s, xt3, x, bcol3, brow)


def _sc_gather(table, idx):
    n, c = table.shape
    total = idx.shape[0]
    info = plsc.get_sparse_core_info()
    nc, ns = info.num_cores, info.num_subcores
    nw = nc * ns
    per_w = total // nw
    cb = 64 if c > 256 else 128
    n_chunks = per_w // cb
    mesh = plsc.VectorSubcoreMesh(core_axis_name="c", subcore_axis_name="s")

    @functools.partial(
        pl.kernel,
        mesh=mesh,
        out_type=jax.ShapeDtypeStruct((total, c), _F32),
        scratch_types=[
            pltpu.VMEM((cb,), _I32),
            pltpu.VMEM((cb, c), _F32),
            pltpu.SemaphoreType.DMA,
        ],
    )
    def gather_k(table_hbm, idx_hbm, out_hbm, idx_v, rows_v, sem):
        wid = lax.axis_index("s") * nc + lax.axis_index("c")
        base = wid * per_w

        def chunk(ci, _):
            off = base + ci * cb
            pltpu.sync_copy(idx_hbm.at[pl.ds(off, cb)], idx_v)
            pltpu.async_copy(table_hbm.at[idx_v], rows_v, sem).wait()
            pltpu.sync_copy(rows_v, out_hbm.at[pl.ds(off, cb)])
            return 0

        lax.fori_loop(0, n_chunks, chunk, 0, unroll=False)

    return gather_k(table, idx)


def _edge_h(x, xg3_kk, wcat_ref, b):
    ef = jnp.concatenate([x - xg3_kk, xg3_kk], axis=1)
    h = lax.dot_general(ef, wcat_ref[...], (((1,), (0,)), ((), ())),
                        preferred_element_type=_F32)
    return h + b


def _stats_body(x_ref, xg_ref, w_ref, b_ref, o_ref):
    g = pl.program_id(0)
    x = x_ref[...]
    b = b_ref[...]
    c = b.shape[1]
    s_acc = jnp.zeros((1, c), _F32)
    q_acc = jnp.zeros((1, c), _F32)
    for kk in range(K):
        h = _edge_h(x, xg_ref[kk], w_ref, b)
        s_acc = s_acc + jnp.sum(h, axis=0, keepdims=True)
        q_acc = q_acc + jnp.sum(h * h, axis=0, keepdims=True)
    upd = jnp.concatenate([s_acc, q_acc, jnp.zeros((6, c), _F32)], axis=0)

    @pl.when(g == 0)
    def _():
        o_ref[...] = jnp.zeros_like(o_ref)

    o_ref[...] += upd


def _stats(x, xg3, wcat, b_row, blk=128):
    n, dp = x.shape
    c = wcat.shape[1]
    return pl.pallas_call(
        _stats_body,
        grid=(n // blk,),
        in_specs=[
            pl.BlockSpec((blk, dp), lambda g: (g, 0)),
            pl.BlockSpec((K, blk, dp), lambda g: (0, g, 0)),
            pl.BlockSpec((2 * dp, c), lambda g: (0, 0)),
            pl.BlockSpec((1, c), lambda g: (0, 0)),
        ],
        out_specs=pl.BlockSpec((8, c), lambda g: (0, 0)),
        out_shape=jax.ShapeDtypeStruct((8, c), _F32),
    )(x, xg3, wcat, b_row)


def _out_body(x_ref, xg_ref, w_ref, b_ref, a_ref, b0_ref, o_ref):
    x = x_ref[...]
    b = b_ref[...]
    a = a_ref[...]
    b0 = b0_ref[...]
    acc = jnp.zeros((x.shape[0], b.shape[1]), _F32)
    for kk in range(K):
        h = _edge_h(x, xg_ref[kk], w_ref, b)
        acc = acc + jnp.maximum(a * h + b0, 0.0)
    o_ref[...] = acc


def _edge_out(x, xg3, wcat, b_row, a_row, b0_row, blk=128):
    n, dp = x.shape
    c = wcat.shape[1]
    return pl.pallas_call(
        _out_body,
        grid=(n // blk,),
        in_specs=[
            pl.BlockSpec((blk, dp), lambda g: (g, 0)),
            pl.BlockSpec((K, blk, dp), lambda g: (0, g, 0)),
            pl.BlockSpec((2 * dp, c), lambda g: (0, 0)),
            pl.BlockSpec((1, c), lambda g: (0, 0)),
            pl.BlockSpec((1, c), lambda g: (0, 0)),
            pl.BlockSpec((1, c), lambda g: (0, 0)),
        ],
        out_specs=pl.BlockSpec((blk, c), lambda g: (g, 0)),
        out_shape=jax.ShapeDtypeStruct((n, c), _F32),
    )(x, xg3, wcat, b_row, a_row, b0_row)


def _final_body(x_ref, b_ref, w1_ref, b1_ref, w2_ref, b2_ref, o_ref,
                pool_scr):
    g = pl.program_id(0)
    ng = pl.num_programs(0)
    x = x_ref[...]
    brow = b_ref[...]

    parts = []
    for cl in range(NUM_CLOUDS):
        m = jnp.where(brow == cl, x, -jnp.inf)
        parts.append(jnp.max(m, axis=0, keepdims=True))
    upd = jnp.concatenate(parts, axis=0)

    @pl.when(g == 0)
    def _():
        pool_scr[...] = jnp.full_like(pool_scr, -jnp.inf)

    pool_scr[...] = jnp.maximum(pool_scr[...], upd)

    @pl.when(g == ng - 1)
    def _():
        pooled = pool_scr[...]
        h = lax.dot_general(pooled, w1_ref[...], (((1,), (0,)), ((), ())),
                            preferred_element_type=_F32)
        h = jnp.maximum(h + b1_ref[...], 0.0)
        o = lax.dot_general(h, w2_ref[...], (((1,), (0,)), ((), ())),
                            preferred_element_type=_F32)
        o_ref[...] = o + b2_ref[...]


def _final(x4, brow, w1t, b1_row, w2t, b2_row, blk=512):
    n, c = x4.shape
    co = w2t.shape[1]
    return pl.pallas_call(
        _final_body,
        grid=(n // blk,),
        in_specs=[
            pl.BlockSpec((blk, c), lambda g: (g, 0)),
            pl.BlockSpec((blk, 1), lambda g: (g, 0)),
            pl.BlockSpec((c, w1t.shape[1]), lambda g: (0, 0)),
            pl.BlockSpec((1, w1t.shape[1]), lambda g: (0, 0)),
            pl.BlockSpec((w2t.shape[0], co), lambda g: (0, 0)),
            pl.BlockSpec((1, co), lambda g: (0, 0)),
        ],
        out_specs=pl.BlockSpec((NUM_CLOUDS, co), lambda g: (0, 0)),
        out_shape=jax.ShapeDtypeStruct((NUM_CLOUDS, co), _F32),
        scratch_shapes=[pltpu.VMEM((NUM_CLOUDS, c), _F32)],
    )(x4, brow, w1t, b1_row, w2t, b2_row)


def _edge_conv(x, bcol, brow, t0s, t1s, W, b, gamma, beta):
    n, d_raw = x.shape
    c = W.shape[0]
    d = max(8, d_raw)
    if d_raw < d:
        x = jnp.pad(x, ((0, 0), (0, d - d_raw)))
    dp = 128
    xp = jnp.pad(x, ((0, 0), (0, dp - d))) if d < dp else x
    wa = W[:, :d_raw]
    wb = W[:, d_raw:]
    wcat = jnp.zeros((2 * dp, c), _F32)
    wcat = wcat.at[:d_raw, :].set(wa.T)
    wcat = wcat.at[dp:dp + d_raw, :].set(wb.T)

    xt3 = x.T.reshape(d, n // TW, TW).transpose(1, 0, 2)
    nbr_pad = _knn(x, xt3, bcol, brow, t0s, t1s,
                   exact_d=d_raw if d_raw <= 8 else 0)

    idx = nbr_pad[:, :K].T.reshape(E)
    xg = _sc_gather(xp, idx)
    xg3 = xg.reshape(K, N, dp)

    b_row = b.reshape(1, c)
    st = _stats(xp, xg3, wcat, b_row)
    mean = st[0] / E
    var = st[1] / E - mean * mean
    scale = gamma / jnp.sqrt(var + 1e-5)
    a_row = scale.reshape(1, c)
    b0_row = (beta - scale * mean).reshape(1, c)
    return _edge_out(xp, xg3, wcat, b_row, a_row, b0_row)


def kernel(pos, batch, W1, b1, g1, be1, W2, b2, g2, be2, W3, b3, g3, be3,
           W4, b4, g4, be4, Wc1, bc1, Wc2, bc2):
    batch = batch.astype(_I32)
    bcol = batch.reshape(N // TW, 1, TW)
    brow = batch.reshape(N, 1)

    starts = jnp.searchsorted(batch, jnp.arange(NUM_CLOUDS + 1, dtype=_I32))
    blk_lo = batch[::RB]
    blk_hi = batch[RB - 1::RB]
    w0 = starts[blk_lo]
    w1 = starts[blk_hi + 1]
    t0s = (w0 // TW).reshape(1, -1).astype(_I32)
    t1s = ((w1 + TW - 1) // TW).reshape(1, -1).astype(_I32)

    x = _edge_conv(pos, bcol, brow, t0s, t1s, W1, b1, g1, be1)
    x = _edge_conv(x, bcol, brow, t0s, t1s, W2, b2, g2, be2)
    x = _edge_conv(x, bcol, brow, t0s, t1s, W3, b3, g3, be3)
    x = _edge_conv(x, bcol, brow, t0s, t1s, W4, b4, g4, be4)

    co_pad = 128
    w2t = jnp.zeros((Wc1.shape[0], co_pad), _F32).at[:, :Wc2.shape[0]].set(Wc2.T)
    b2_row = jnp.zeros((1, co_pad), _F32).at[0, :Wc2.shape[0]].set(bc2)
    out_pad = _final(x, brow, Wc1.T, bc1.reshape(1, -1), w2t, b2_row)
    return out_pad[:, :Wc2.shape[0]]
